# baseline (speedup 1.0000x reference)
import jax
import jax.numpy as jnp
from jax.experimental import pallas as pl

_N_HEADS = 16
_HEAD_DIM = 128
_ROPE_DIM = 64
_TOPK = 1024
_LN_EPS = 1e-5
_W_SCALE = 0.022097086912079608
_INV_HD = 0.0078125


def _lane_iota(shape):
    return jax.lax.broadcasted_iota(jnp.int32, shape, 1)


def _rope_blocks(x, cos, sin):
    rows, cols = x.shape
    nb = cols // _HEAD_DIM
    c = cos[:, :32]
    s = sin[:, :32]
    cpat = jnp.concatenate([c, c, c, c], axis=1)
    spat = jnp.concatenate([s, s, s, s], axis=1)
    if nb > 1:
        cpat = jnp.concatenate([cpat] * nb, axis=1)
        spat = jnp.concatenate([spat] * nb, axis=1)
    j = _lane_iota(x.shape) % _HEAD_DIM
    x_m32 = jnp.roll(x, 32, axis=1)
    x_p32 = jnp.roll(x, -32, axis=1)
    first = x * cpat - x_p32 * spat
    second = x_m32 * spat + x * cpat
    return jnp.where(j < 32, first, jnp.where(j < 64, second, x))


def _hadamard_blocks(x, scale):
    j = _lane_iota(x.shape)
    h = 1
    while h < _HEAD_DIM:
        upper = (j // h) % 2 == 1
        x_p = jnp.roll(x, -h, axis=1)
        x_m = jnp.roll(x, h, axis=1)
        x = jnp.where(upper, x_m - x, x + x_p)
        h *= 2
    return x * scale


def _row_sum_tree(x):
    acc = x[:, 0:8]
    for c in range(1, 16):
        acc = acc + x[:, 8 * c:8 * c + 8]
    b = acc[:, 0:4] + acc[:, 4:8]
    b = b[:, 0:2] + b[:, 2:4]
    return b[:, 0:1] + b[:, 1:2]


def _k_kernel(hid_ref, wk_ref, gamma_ref, beta_ref, cos_ref, sin_ref, ww_ref,
              kout_ref, wout_ref):
    kt = jax.lax.dot_general(wk_ref[...], hid_ref[...], (((0,), (1,)), ((), ())),
                             preferred_element_type=jnp.float32)
    k = kt.swapaxes(0, 1)
    mu = _row_sum_tree(k) * _INV_HD
    d = k - mu
    var = _row_sum_tree(d * d) * _INV_HD
    k = d / jnp.sqrt(var + _LN_EPS) * gamma_ref[...] + beta_ref[...]
    k = _rope_blocks(k, cos_ref[...], sin_ref[...])
    kout_ref[...] = _hadamard_blocks(k, _HEAD_DIM ** (-0.5))
    wt = jax.lax.dot_general(ww_ref[...], hid_ref[...], (((0,), (1,)), ((), ())),
                             preferred_element_type=jnp.float32)
    wout_ref[...] = (wt * _W_SCALE).swapaxes(0, 1)


def _bitonic_topk(v):
    rows, width = v.shape
    lane = _lane_iota(v.shape)
    idx = lane
    k = 2
    while k <= width:
        j = k // 2
        while j >= 1:
            bit_j = (lane & j) != 0
            bit_k = (lane & k) != 0
            pv = jnp.where(bit_j, jnp.roll(v, j, axis=1), jnp.roll(v, -j, axis=1))
            pi = jnp.where(bit_j, jnp.roll(idx, j, axis=1), jnp.roll(idx, -j, axis=1))
            before = (v > pv) | ((v == pv) & (idx < pi))
            take = before ^ (~bit_j) ^ bit_k
            v = jnp.where(take, pv, v)
            idx = jnp.where(take, pi, idx)
            j //= 2
        k *= 2
    return v[:, :_TOPK], idx[:, :_TOPK]


def _score_kernel(qlora_ref, wqb_ref, cos_ref, sin_ref, kfin_ref, w2_ref,
                  out_ref):
    qt = jax.lax.dot_general(wqb_ref[...], qlora_ref[...],
                             (((0,), (1,)), ((), ())),
                             preferred_element_type=jnp.float32)
    q = qt.swapaxes(0, 1)
    q = _rope_blocks(q, cos_ref[...], sin_ref[...])
    q = _hadamard_blocks(q, _HEAD_DIM ** (-0.5))
    kfin = kfin_ref[...]
    w2 = w2_ref[...]
    acc = None
    for h in range(_N_HEADS):
        qh = q[:, h * _HEAD_DIM:(h + 1) * _HEAD_DIM]
        d = jax.lax.dot_general(qh, kfin, (((1,), (1,)), ((), ())),
                                preferred_element_type=jnp.float32)
        t = jnp.maximum(d, 0.0) * w2[:, h:h + 1]
        acc = t if acc is None else acc + t
    out_ref[...] = acc + 0.0


def _topk_kernel(sc_ref, val_ref, idx_ref):
    values, indices = _bitonic_topk(sc_ref[...])
    val_ref[...] = values
    idx_ref[...] = indices


def _index_scores(hidden_states, q_lora, cos, sin, wq_b, wk, k_gamma, k_beta,
                  weights_w):
    s = hidden_states.shape[1]
    hid = hidden_states[0]
    qlora = q_lora[0]
    cos2 = cos[0]
    sin2 = sin[0]

    k_final, w2 = pl.pallas_call(
        _k_kernel,
        in_specs=[pl.BlockSpec(hid.shape, lambda: (0, 0)),
                  pl.BlockSpec(wk.shape, lambda: (0, 0)),
                  pl.BlockSpec((1, _HEAD_DIM), lambda: (0, 0)),
                  pl.BlockSpec((1, _HEAD_DIM), lambda: (0, 0)),
                  pl.BlockSpec(cos2.shape, lambda: (0, 0)),
                  pl.BlockSpec(sin2.shape, lambda: (0, 0)),
                  pl.BlockSpec(weights_w.shape, lambda: (0, 0))],
        out_specs=[pl.BlockSpec((s, _HEAD_DIM), lambda: (0, 0)),
                   pl.BlockSpec((s, _N_HEADS), lambda: (0, 0))],
        out_shape=[jax.ShapeDtypeStruct((s, _HEAD_DIM), jnp.float32),
                   jax.ShapeDtypeStruct((s, _N_HEADS), jnp.float32)],
    )(hid, wk, k_gamma.reshape(1, -1), k_beta.reshape(1, -1), cos2, sin2,
      weights_w)

    tq = 256
    scores = pl.pallas_call(
        _score_kernel,
        grid=(s // tq,),
        in_specs=[
            pl.BlockSpec((tq, qlora.shape[1]), lambda i: (i, 0)),
            pl.BlockSpec((qlora.shape[1], _N_HEADS * _HEAD_DIM), lambda i: (0, 0)),
            pl.BlockSpec((tq, _ROPE_DIM), lambda i: (i, 0)),
            pl.BlockSpec((tq, _ROPE_DIM), lambda i: (i, 0)),
            pl.BlockSpec((s, _HEAD_DIM), lambda i: (0, 0)),
            pl.BlockSpec((tq, _N_HEADS), lambda i: (i, 0)),
        ],
        out_specs=pl.BlockSpec((tq, s), lambda i: (i, 0)),
        out_shape=jax.ShapeDtypeStruct((s, s), jnp.float32),
    )(qlora, wq_b, cos2, sin2, k_final, w2)

    tr = 8
    values, indices = pl.pallas_call(
        _topk_kernel,
        grid=(s // tr,),
        in_specs=[pl.BlockSpec((tr, s), lambda i: (i, 0))],
        out_specs=[pl.BlockSpec((tr, _TOPK), lambda i: (i, 0)),
                   pl.BlockSpec((tr, _TOPK), lambda i: (i, 0))],
        out_shape=[jax.ShapeDtypeStruct((s, _TOPK), jnp.float32),
                   jax.ShapeDtypeStruct((s, _TOPK), jnp.int32)],
    )(scores)
    return values, indices


def kernel(hidden_states, q_lora, cos, sin, attention_mask, wq_b, wk, k_gamma,
           k_beta, weights_w):
    values, indices = _index_scores(hidden_states, q_lora, cos, sin, wq_b, wk,
                                    k_gamma, k_beta, weights_w)
    return values[None], indices[None]
